# Initial kernel scaffold; baseline (speedup 1.0000x reference)
import jax
import jax.numpy as jnp
from jax.experimental import pallas as pl


def kernel(input_ids, word_embeddings, ln_weight, ln_bias):
    raise NotImplementedError("write your pallas kernel here")


# speedup vs baseline: 1.6244x; 1.6244x over previous
import functools

import numpy as np
import jax
import jax.numpy as jnp
from jax import lax
from jax.experimental import pallas as pl
from jax.experimental.pallas import tpu as pltpu
from jax.experimental.pallas import tpu_sc as plsc

_VOCAB = 100000
_HID = 128
_PAD = 0
_MAXPOS = 16384
_BIAS = 1
_EPS = 1e-12

_NW = 32
_BLK = 128


def _sinusoidal_table(num_embeddings, embedding_dim, padding_idx):
    half_dim = embedding_dim // 2
    emb = np.log(10000.0) / (half_dim - 1)
    emb = np.exp(np.arange(half_dim, dtype=np.float64) * -emb)
    emb = np.arange(num_embeddings, dtype=np.float64)[:, None] * emb[None, :]
    table = np.concatenate([np.sin(emb), np.cos(emb)], axis=1)
    if embedding_dim % 2 == 1:
        table = np.concatenate([table, np.zeros((num_embeddings, 1))], axis=1)
    if padding_idx is not None:
        table[padding_idx, :] = 0.0
    return np.asarray(table, dtype=np.float32)


_POS_TABLE = _sinusoidal_table(_MAXPOS, _HID, _PAD)


def _sc_body(S, CHUNK, ROW_W,
             ids_hbm, wtab_hbm, ptab_hbm, lnw_hbm, lnb_hbm, out_hbm,
             row_ids, pos_idx, widx, pidx, wbuf, pbuf, lnw_v, lnb_v,
             sem_w, sem_p):
    wid = lax.axis_index("c") * 16 + lax.axis_index("s")
    b = wid // ROW_W
    c = wid % ROW_W

    pltpu.sync_copy(ids_hbm.at[pl.ds(b * S, S)], row_ids)
    pltpu.sync_copy(lnw_hbm, lnw_v)
    pltpu.sync_copy(lnb_hbm, lnb_v)

    nv = c * (CHUNK // 16)

    def pc_body(j, acc):
        v = row_ids[pl.ds(j * 16, 16)]
        return acc + jnp.sum(jnp.minimum(v, 1))

    cnt = lax.fori_loop(0, nv, pc_body, jnp.int32(0))

    def pos_body(j, cnt):
        v = row_ids[pl.ds(c * CHUNK + j * 16, 16)]
        mi = jnp.minimum(v, 1)
        cs = plsc.cumsum(mi)
        cv = jnp.full((16,), cnt, jnp.int32)
        pos_idx[pl.ds(j * 16, 16)] = (cv + cs) * mi + (_PAD + _BIAS)
        return cnt + jnp.sum(mi)

    lax.fori_loop(0, CHUNK // 16, pos_body, cnt)

    lnw_regs = [lnw_v[pl.ds(k * 16, 16)] for k in range(_HID // 16)]
    lnb_regs = [lnb_v[pl.ds(k * 16, 16)] for k in range(_HID // 16)]

    for blk in range(CHUNK // _BLK):
        for j in range(_BLK // 16):
            widx[pl.ds(j * 16, 16)] = row_ids[pl.ds(c * CHUNK + blk * _BLK + j * 16, 16)]
            pidx[pl.ds(j * 16, 16)] = pos_idx[pl.ds(blk * _BLK + j * 16, 16)]
        cp_w = pltpu.async_copy(wtab_hbm.at[widx], wbuf, sem_w)
        cp_p = pltpu.async_copy(ptab_hbm.at[pidx], pbuf, sem_p)
        cp_w.wait()
        cp_p.wait()

        def tok_body(t, carry):
            es = []
            s = jnp.zeros((16,), jnp.float32)
            q = jnp.zeros((16,), jnp.float32)
            for k in range(_HID // 16):
                e = wbuf[t, pl.ds(k * 16, 16)] + pbuf[t, pl.ds(k * 16, 16)]
                es.append(e)
                s = s + e
                q = q + e * e
            mean = jnp.sum(s) * (1.0 / _HID)
            ex2 = jnp.sum(q) * (1.0 / _HID)
            var = ex2 - mean * mean
            xv = jnp.full((16,), var + _EPS, jnp.float32)
            bits = lax.bitcast_convert_type(xv, jnp.int32)
            y = lax.bitcast_convert_type(
                jnp.int32(0x5F3759DF) - lax.shift_right_logical(bits, 1),
                jnp.float32)
            for _ in range(4):
                y = y * (1.5 - 0.5 * xv * y * y)
            mv = jnp.full((16,), mean, jnp.float32)
            for k in range(_HID // 16):
                wbuf[t, pl.ds(k * 16, 16)] = (
                    (es[k] - mv) * y * lnw_regs[k] + lnb_regs[k])
            return carry

        lax.fori_loop(0, _BLK, tok_body, 0)
        pltpu.sync_copy(wbuf, out_hbm.at[pl.ds(wid * CHUNK + blk * _BLK, _BLK)])


def kernel(input_ids, word_embeddings, ln_weight, ln_bias):
    B, S = input_ids.shape
    HID = word_embeddings.shape[1]
    TOK = B * S
    CHUNK = TOK // _NW
    ROW_W = S // CHUNK

    mesh = plsc.VectorSubcoreMesh(core_axis_name="c", subcore_axis_name="s")
    run = functools.partial(
        pl.kernel,
        out_type=jax.ShapeDtypeStruct((TOK, HID), jnp.float32),
        mesh=mesh,
        compiler_params=pltpu.CompilerParams(needs_layout_passes=False),
        scratch_types=[
            pltpu.VMEM((S,), jnp.int32),
            pltpu.VMEM((CHUNK,), jnp.int32),
            pltpu.VMEM((_BLK,), jnp.int32),
            pltpu.VMEM((_BLK,), jnp.int32),
            pltpu.VMEM((_BLK, HID), jnp.float32),
            pltpu.VMEM((_BLK, HID), jnp.float32),
            pltpu.VMEM((HID,), jnp.float32),
            pltpu.VMEM((HID,), jnp.float32),
            pltpu.SemaphoreType.DMA,
            pltpu.SemaphoreType.DMA,
        ],
    )(functools.partial(_sc_body, S, CHUNK, ROW_W))

    pos_tab = jnp.asarray(_POS_TABLE)
    out = run(input_ids.reshape(-1), word_embeddings, pos_tab,
              ln_weight, ln_bias)
    return out.reshape(B, S, HID)
